# baseline (device time: 18486 ns/iter reference)
import jax
import jax.numpy as jnp
from jax import lax
from jax.experimental import pallas as pl
from jax.experimental.pallas import tpu as pltpu

K = 16
GRP = 16
_NEG = float("-inf")


def _topk_rows_desc(vals, k):
    m = jnp.max(vals, axis=1, keepdims=True)
    outs = [m]
    for _ in range(k - 1):
        m = jnp.max(jnp.where(vals < m, vals, _NEG), axis=1, keepdims=True)
        outs.append(m)
    return jnp.concatenate(outs, axis=1)


def _local_topk(v, k):
    rows, cols = v.shape
    v3 = v.reshape(rows, GRP, cols // GRP)
    m1 = jnp.max(v3, axis=1)
    m2 = jnp.max(jnp.where(v3 < m1[:, None, :], v3, _NEG), axis=1)
    cands = jnp.concatenate([m1, m2], axis=1)
    return _topk_rows_desc(cands, k)


def kernel(x):
    m, n = x.shape

    def body(x_hbm, out_ref, xin, cand_ref, in_sem, send_sem, recv_sem):
        my_x = lax.axis_index("x")
        my_y = lax.axis_index("y")
        my_z = lax.axis_index("z")
        peer = (1 - my_x, my_y, my_z)

        cp = pltpu.make_async_copy(x_hbm, xin, in_sem)
        cp.start()

        barrier_sem = pltpu.get_barrier_semaphore()
        pl.semaphore_signal(
            barrier_sem, inc=1,
            device_id=peer, device_id_type=pl.DeviceIdType.MESH,
        )
        pl.semaphore_wait(barrier_sem, 1)
        cp.wait()

        cand_ref[0, :, :] = _local_topk(xin[:, :].astype(jnp.float32), K)

        rdma = pltpu.make_async_remote_copy(
            src_ref=cand_ref.at[0],
            dst_ref=cand_ref.at[1],
            send_sem=send_sem,
            recv_sem=recv_sem,
            device_id=peer,
            device_id_type=pl.DeviceIdType.MESH,
        )
        rdma.start()
        rdma.wait_recv()

        both = jnp.concatenate([cand_ref[0, :, :], cand_ref[1, :, :]], axis=1)
        out_ref[:, :] = _topk_rows_desc(both, K)
        rdma.wait_send()

    return pl.pallas_call(
        body,
        out_shape=jax.ShapeDtypeStruct((m, K), jnp.float32),
        in_specs=[pl.BlockSpec(memory_space=pl.ANY)],
        out_specs=pl.BlockSpec(memory_space=pltpu.VMEM),
        scratch_shapes=[
            pltpu.VMEM((m, n), x.dtype),
            pltpu.VMEM((2, m, K), jnp.float32),
            pltpu.SemaphoreType.DMA,
            pltpu.SemaphoreType.DMA,
            pltpu.SemaphoreType.DMA,
        ],
        compiler_params=pltpu.CompilerParams(collective_id=0),
    )(x)


# device time: 17897 ns/iter; 1.0329x vs baseline; 1.0329x over previous
import jax
import jax.numpy as jnp
from jax import lax
from jax.experimental import pallas as pl
from jax.experimental.pallas import tpu as pltpu

K = 16
GRP = 16
_NEG = float("-inf")


def _topk_rows_desc(vals, k):
    m = jnp.max(vals, axis=1, keepdims=True)
    outs = [m]
    for _ in range(k - 1):
        m = jnp.max(jnp.where(vals < m, vals, _NEG), axis=1, keepdims=True)
        outs.append(m)
    return jnp.concatenate(outs, axis=1)


def _local_topk(v, k):
    rows, cols = v.shape
    v3 = v.reshape(rows, GRP, cols // GRP)
    m1 = jnp.max(v3, axis=1)
    m2 = jnp.max(jnp.where(v3 < m1[:, None, :], v3, _NEG), axis=1)
    cands = jnp.concatenate([m1, m2], axis=1)
    return _topk_rows_desc(cands, k)


def kernel(x):
    m, n = x.shape

    def body(x_ref, out_ref, cand_ref, send_sem, recv_sem):
        my_x = lax.axis_index("x")
        my_y = lax.axis_index("y")
        my_z = lax.axis_index("z")
        peer = (1 - my_x, my_y, my_z)

        barrier_sem = pltpu.get_barrier_semaphore()
        pl.semaphore_signal(
            barrier_sem, inc=1,
            device_id=peer, device_id_type=pl.DeviceIdType.MESH,
        )

        cand_ref[0, :, :] = _local_topk(x_ref[:, :].astype(jnp.float32), K)

        pl.semaphore_wait(barrier_sem, 1)

        rdma = pltpu.make_async_remote_copy(
            src_ref=cand_ref.at[0],
            dst_ref=cand_ref.at[1],
            send_sem=send_sem,
            recv_sem=recv_sem,
            device_id=peer,
            device_id_type=pl.DeviceIdType.MESH,
        )
        rdma.start()
        rdma.wait_recv()

        both = jnp.concatenate([cand_ref[0, :, :], cand_ref[1, :, :]], axis=1)
        out_ref[:, :] = _topk_rows_desc(both, K)
        rdma.wait_send()

    return pl.pallas_call(
        body,
        out_shape=jax.ShapeDtypeStruct((m, K), jnp.float32),
        in_specs=[pl.BlockSpec(memory_space=pltpu.VMEM)],
        out_specs=pl.BlockSpec(memory_space=pltpu.VMEM),
        scratch_shapes=[
            pltpu.VMEM((2, m, K), jnp.float32),
            pltpu.SemaphoreType.DMA,
            pltpu.SemaphoreType.DMA,
        ],
        compiler_params=pltpu.CompilerParams(collective_id=0),
    )(x)


# device time: 16309 ns/iter; 1.1335x vs baseline; 1.0974x over previous
import jax
import jax.numpy as jnp
from jax import lax
from jax.experimental import pallas as pl
from jax.experimental.pallas import tpu as pltpu

K = 16
GRP = 32
_NEG = float("-inf")


def _topk_rows_desc(vals, k):
    m = jnp.max(vals, axis=1, keepdims=True)
    outs = [m]
    for _ in range(k - 1):
        m = jnp.max(jnp.where(vals < m, vals, _NEG), axis=1, keepdims=True)
        outs.append(m)
    return jnp.concatenate(outs, axis=1)


def _local_topk(v, k):
    rows, cols = v.shape
    v3 = v.reshape(rows, GRP, cols // GRP)
    m1 = jnp.max(v3, axis=1)
    m2 = jnp.max(jnp.where(v3 < m1[:, None, :], v3, _NEG), axis=1)
    cands = jnp.concatenate([m1, m2], axis=1)
    return _topk_rows_desc(cands, k)


def kernel(x):
    m, n = x.shape

    def body(x_ref, out_ref, cand_ref, send_sem, recv_sem):
        my_x = lax.axis_index("x")
        my_y = lax.axis_index("y")
        my_z = lax.axis_index("z")
        peer = (1 - my_x, my_y, my_z)

        barrier_sem = pltpu.get_barrier_semaphore()
        pl.semaphore_signal(
            barrier_sem, inc=1,
            device_id=peer, device_id_type=pl.DeviceIdType.MESH,
        )

        cand_ref[0, :, :] = _local_topk(x_ref[:, :].astype(jnp.float32), K)

        pl.semaphore_wait(barrier_sem, 1)

        rdma = pltpu.make_async_remote_copy(
            src_ref=cand_ref.at[0],
            dst_ref=cand_ref.at[1],
            send_sem=send_sem,
            recv_sem=recv_sem,
            device_id=peer,
            device_id_type=pl.DeviceIdType.MESH,
        )
        rdma.start()
        rdma.wait_recv()

        both = jnp.concatenate([cand_ref[0, :, :], cand_ref[1, :, :]], axis=1)
        out_ref[:, :] = _topk_rows_desc(both, K)
        rdma.wait_send()

    return pl.pallas_call(
        body,
        out_shape=jax.ShapeDtypeStruct((m, K), jnp.float32),
        in_specs=[pl.BlockSpec(memory_space=pltpu.VMEM)],
        out_specs=pl.BlockSpec(memory_space=pltpu.VMEM),
        scratch_shapes=[
            pltpu.VMEM((2, m, K), jnp.float32),
            pltpu.SemaphoreType.DMA,
            pltpu.SemaphoreType.DMA,
        ],
        compiler_params=pltpu.CompilerParams(collective_id=0),
    )(x)
